# Initial kernel scaffold; baseline (speedup 1.0000x reference)
import jax
import jax.numpy as jnp
from jax.experimental import pallas as pl


def kernel(src_token_ids, dst_token_ids, edge_index, emb, W_ih, W_hh, b_ih, b_hh, ln_g, ln_b, W_fc, b_fc):
    raise NotImplementedError("write your pallas kernel here")


# speedup vs baseline: 2.4544x; 2.4544x over previous
import dataclasses
import functools

import jax
import jax.numpy as jnp
from jax import lax
from jax.experimental import pallas as pl
from jax.experimental.pallas import tpu as pltpu
from jax.experimental.pallas import tpu_sc as plsc

N = 10000
K = 32
D = 128
C = 10

NUM_TILES = 32
CH = 4
ECH = CH * K
NCHUNK = N // CH
TPT = -(-NCHUNK // NUM_TILES)


def _sc_gather_reduce(src_tok, esrc, emb):
    mesh = plsc.VectorSubcoreMesh(core_axis_name="c", subcore_axis_name="s")
    cp = pltpu.CompilerParams()
    if "needs_layout_passes" in pltpu.CompilerParams.__dataclass_fields__:
        cp = dataclasses.replace(cp, needs_layout_passes=False)

    @functools.partial(
        pl.kernel,
        compiler_params=cp,
        out_type=[
            jax.ShapeDtypeStruct((N, D), jnp.float32),
            jax.ShapeDtypeStruct((N, D), jnp.float32),
        ],
        mesh=mesh,
        scratch_types=[
            pltpu.VMEM((N,), jnp.int32),
            pltpu.VMEM((ECH,), jnp.int32),
            pltpu.VMEM((ECH,), jnp.int32),
            pltpu.VMEM((ECH, D), jnp.float32),
            pltpu.VMEM((CH, D), jnp.float32),
            pltpu.VMEM((CH, D), jnp.float32),
            pltpu.SemaphoreType.DMA,
        ],
    )
    def sc_kernel(tok_hbm, esrc_hbm, emb_hbm, total_hbm, xlast_hbm,
                  tok_v, eidx_v, tokidx_v, rows_v, tot_v, xl_v, sem):
        wid = lax.axis_index("s") * 2 + lax.axis_index("c")
        pltpu.sync_copy(tok_hbm, tok_v)

        def chunk_body(t, carry):
            c = wid + NUM_TILES * t

            @pl.when(c < NCHUNK)
            def _():
                pltpu.sync_copy(esrc_hbm.at[pl.ds(c * ECH, ECH)], eidx_v)
                for i in range(ECH // 16):
                    sl = pl.ds(i * 16, 16)
                    tokidx_v[sl] = plsc.load_gather(tok_v, [eidx_v[sl]])
                pltpu.async_copy(emb_hbm.at[tokidx_v], rows_v, sem).wait()
                for j in range(CH):
                    for cb in range(D // 16):
                        sl = pl.ds(cb * 16, 16)
                        acc = rows_v[j * K, sl]
                        for k in range(1, K):
                            acc = acc + rows_v[j * K + k, sl]
                        tot_v[j, sl] = acc
                        xl_v[j, sl] = rows_v[j * K + K - 1, sl]
                pltpu.sync_copy(tot_v, total_hbm.at[pl.ds(c * CH, CH)])
                pltpu.sync_copy(xl_v, xlast_hbm.at[pl.ds(c * CH, CH)])

            return carry

        lax.fori_loop(0, TPT, chunk_body, 0)

    return sc_kernel(src_tok, esrc, emb)


def _tc_body(x_ref, tot_ref, wih_ref, whh_ref, bih_ref, bhh_ref,
             lng_ref, lnb_ref, wfc_ref, bfc_ref, o_ref):
    x = x_ref[...]
    h = tot_ref[...] - x
    gi = jnp.dot(x, wih_ref[...], preferred_element_type=jnp.float32) + bih_ref[...]
    gh = jnp.dot(h, whh_ref[...], preferred_element_type=jnp.float32) + bhh_ref[...]
    r = jax.nn.sigmoid(gi[:, :D] + gh[:, :D])
    z = jax.nn.sigmoid(gi[:, D:2 * D] + gh[:, D:2 * D])
    n = jnp.tanh(gi[:, 2 * D:] + r * gh[:, 2 * D:])
    ho = (1.0 - z) * n + z * h
    mu = jnp.mean(ho, axis=-1, keepdims=True)
    var = jnp.mean((ho - mu) ** 2, axis=-1, keepdims=True)
    rst = lng_ref[...] * (ho - mu) / jnp.sqrt(var + 1e-5) + lnb_ref[...]
    o_ref[...] = jnp.dot(rst, wfc_ref[...], preferred_element_type=jnp.float32) + bfc_ref[...]


def _tc_head(x, total, W_ihT, W_hhT, b_ih2, b_hh2, ln_g2, ln_b2, W_fcT, b_fc2):
    B = 2000
    grid = (N // B,)
    full = lambda shape: pl.BlockSpec(shape, lambda i: (0, 0))
    return pl.pallas_call(
        _tc_body,
        grid=grid,
        in_specs=[
            pl.BlockSpec((B, D), lambda i: (i, 0)),
            pl.BlockSpec((B, D), lambda i: (i, 0)),
            full((D, 3 * D)),
            full((D, 3 * D)),
            full((1, 3 * D)),
            full((1, 3 * D)),
            full((1, D)),
            full((1, D)),
            full((D, D)),
            full((1, D)),
        ],
        out_specs=pl.BlockSpec((B, D), lambda i: (i, 0)),
        out_shape=jax.ShapeDtypeStruct((N, D), jnp.float32),
    )(x, total, W_ihT, W_hhT, b_ih2, b_hh2, ln_g2, ln_b2, W_fcT, b_fc2)


def kernel(src_token_ids, dst_token_ids, edge_index, emb, W_ih, W_hh,
           b_ih, b_hh, ln_g, ln_b, W_fc, b_fc):
    src_tok = src_token_ids.astype(jnp.int32)
    esrc = edge_index[0].astype(jnp.int32)
    total, xlast = _sc_gather_reduce(src_tok, esrc, emb)

    W_ihT = W_ih.T
    W_hhT = W_hh.T
    b_ih2 = b_ih.reshape(1, 3 * D)
    b_hh2 = b_hh.reshape(1, 3 * D)
    ln_g2 = ln_g.reshape(1, D)
    ln_b2 = ln_b.reshape(1, D)
    W_fcT = jnp.pad(W_fc.T, ((0, 0), (0, D - C)))
    b_fc2 = jnp.pad(b_fc, (0, D - C)).reshape(1, D)

    out_pad = _tc_head(xlast, total, W_ihT, W_hhT, b_ih2, b_hh2,
                       ln_g2, ln_b2, W_fcT, b_fc2)
    return out_pad[:, :C]
